# baseline (speedup 1.0000x reference)
import functools

import jax
import jax.numpy as jnp
from jax import lax
from jax.experimental import pallas as pl
from jax.experimental.pallas import tpu as pltpu
from jax.experimental.pallas import tpu_sc as plsc

B = 4096
K = 16
DIM = 1024

NC = 2
NS = 16
NW = NC * NS
L = 16

BPW = B // NW
CH = 16
NCH = BPW // CH
VPR = DIM // L
NBUF = 2
NACC = 4

_mesh = plsc.VectorSubcoreMesh(core_axis_name="c", subcore_axis_name="s")


@functools.partial(
    pl.kernel,
    out_type=jax.ShapeDtypeStruct((NW, L), jnp.float32),
    mesh=_mesh,
    scratch_types=[
        pltpu.VMEM((BPW,), jnp.int32),
        pltpu.VMEM((BPW,), jnp.int32),
        pltpu.VMEM((NBUF, CH, DIM), jnp.float32),
        pltpu.VMEM((NBUF * CH * DIM // 2,), jnp.int32),
        pltpu.VMEM((L,), jnp.float32),
        [pltpu.SemaphoreType.DMA] * NBUF,
        [pltpu.SemaphoreType.DMA] * NBUF,
    ],
)
def _sc_partial_sums(in_hbm, dec_hbm, lbl_hbm, out_hbm,
                     lbl_v, idx_v, dec_buf, in_buf, acc_buf, sg, si):
    wid = lax.axis_index("s") * NC + lax.axis_index("c")
    base = wid * BPW

    HW = DIM // 2

    def start_in(c, s):
        pltpu.async_copy(
            in_hbm.at[pl.ds((base + c * CH) * HW, CH * HW)],
            in_buf.at[pl.ds(s * CH * HW, CH * HW)], si[s])

    start_in(0, 0)
    start_in(1, 1)

    pltpu.sync_copy(lbl_hbm.at[pl.ds(base, BPW)], lbl_v)
    lane = lax.iota(jnp.int32, L)
    for c in range(BPW // L):
        lbl = lbl_v[pl.ds(c * L, L)]
        idx_v[pl.ds(c * L, L)] = (base + c * L) * K + lane * K + lbl

    def start_g(c, s):
        pltpu.async_copy(
            dec_hbm.at[idx_v.at[pl.ds(c * CH, CH)]], dec_buf.at[s], sg[s])

    def start(c, s):
        start_g(c, s)
        start_in(c, s)

    def wait(c, s):
        pltpu.make_async_copy(
            dec_hbm.at[idx_v.at[pl.ds(c * CH, CH)]], dec_buf.at[s],
            sg[s]).wait()
        pltpu.make_async_copy(
            in_hbm.at[pl.ds((base + c * CH) * HW, CH * HW)],
            in_buf.at[pl.ds(s * CH * HW, CH * HW)], si[s]).wait()

    def compute(s, acc):
        def row_body(r, acc):
            for g in range(DIM // 32):
                v = in_buf[pl.ds(s * CH * HW + r * HW + g * L, L)]
                xa = lax.bitcast_convert_type(
                    lax.shift_left(v, 16), jnp.float32)
                xb = lax.bitcast_convert_type(
                    jnp.bitwise_and(v, jnp.int32(-65536)), jnp.float32)
                e0 = xa - dec_buf[s, r, pl.ds(g * 32, L)]
                e1 = xb - dec_buf[s, r, pl.ds(g * 32 + L, L)]
                acc = acc + e0 * e0
                acc = acc + e1 * e1
            return acc

        return lax.fori_loop(0, CH, row_body, acc)

    start_g(0, 0)
    start_g(1, 1)

    def pair_body(t, acc):
        c0 = 2 * t
        wait(c0, 0)
        acc = compute(0, acc)

        @pl.when(t + 1 < NCH // 2)
        def _():
            start(c0 + 2, 0)

        wait(c0 + 1, 1)
        acc = compute(1, acc)

        @pl.when(t + 1 < NCH // 2)
        def _():
            start(c0 + 3, 1)

        return acc

    acc = lax.fori_loop(0, NCH // 2, pair_body,
                        jnp.zeros((L,), jnp.float32))

    acc_buf[...] = acc
    pltpu.sync_copy(acc_buf, out_hbm.at[wid])


def _tc_finish_body(p_ref, o_ref):
    o_ref[0, 0] = jnp.sqrt(jnp.sum(p_ref[...])) / B


_tc_finish = pl.pallas_call(
    _tc_finish_body,
    out_shape=jax.ShapeDtypeStruct((1, 1), jnp.float32),
    out_specs=pl.BlockSpec(memory_space=pltpu.SMEM),
)


def kernel(inputs, decoded, labels):
    dec_flat = decoded.reshape(B * K, DIM)
    lbl = labels.astype(jnp.int32)
    x_pairs = (inputs.reshape(B, DIM // 32, 2, 16)
               .swapaxes(2, 3)
               .astype(jnp.bfloat16))
    x_packed = lax.bitcast_convert_type(x_pairs, jnp.int32)
    x_packed = x_packed.reshape(B * DIM // 2)
    partials = _sc_partial_sums(x_packed, dec_flat, lbl)
    return _tc_finish(partials)[0, 0]


# speedup vs baseline: 2.0163x; 2.0163x over previous
import functools

import jax
import jax.numpy as jnp
from jax import lax
from jax.experimental import pallas as pl
from jax.experimental.pallas import tpu as pltpu
from jax.experimental.pallas import tpu_sc as plsc

B = 4096
K = 16
DIM = 1024

NC = 2
NS = 16
NW = NC * NS
L = 16

BPW = B // NW
CH = 16
NCH = BPW // CH
VPR = DIM // L
NBUF = 2
NACC = 4

_mesh = plsc.VectorSubcoreMesh(core_axis_name="c", subcore_axis_name="s")


@functools.partial(
    pl.kernel,
    out_type=jax.ShapeDtypeStruct((NW, L), jnp.float32),
    mesh=_mesh,
    scratch_types=[
        pltpu.VMEM((BPW,), jnp.int32),
        pltpu.VMEM((BPW,), jnp.int32),
        pltpu.VMEM((NBUF, CH, DIM), jnp.float32),
        pltpu.VMEM((NBUF * CH * DIM // 2,), jnp.int32),
        pltpu.VMEM((L,), jnp.float32),
        [pltpu.SemaphoreType.DMA] * NBUF,
        [pltpu.SemaphoreType.DMA] * NBUF,
    ],
)
def _sc_partial_sums(in_hbm, dec_hbm, lbl_hbm, out_hbm,
                     lbl_v, idx_v, dec_buf, in_buf, acc_buf, sg, si):
    wid = lax.axis_index("s") * NC + lax.axis_index("c")
    base = wid * BPW

    HW = DIM // 2

    def start_in(c, s):
        pltpu.async_copy(
            in_hbm.at[pl.ds((base + c * CH) * HW, CH * HW)],
            in_buf.at[pl.ds(s * CH * HW, CH * HW)], si[s])

    start_in(0, 0)
    start_in(1, 1)

    pltpu.sync_copy(lbl_hbm.at[pl.ds(base, BPW)], lbl_v)
    lane = lax.iota(jnp.int32, L)
    for c in range(BPW // L):
        lbl = lbl_v[pl.ds(c * L, L)]
        idx_v[pl.ds(c * L, L)] = (base + c * L) * K + lane * K + lbl

    def start_g(c, s):
        pltpu.async_copy(
            dec_hbm.at[idx_v.at[pl.ds(c * CH, CH)]], dec_buf.at[s], sg[s])

    def start(c, s):
        start_g(c, s)
        start_in(c, s)

    def wait(c, s):
        pltpu.make_async_copy(
            dec_hbm.at[idx_v.at[pl.ds(c * CH, CH)]], dec_buf.at[s],
            sg[s]).wait()
        pltpu.make_async_copy(
            in_hbm.at[pl.ds((base + c * CH) * HW, CH * HW)],
            in_buf.at[pl.ds(s * CH * HW, CH * HW)], si[s]).wait()

    def compute(s, acc):
        def row_body(r, acc):
            for g in range(HW // L):
                v = in_buf[pl.ds(s * CH * HW + r * HW + g * L, L)]
                xa = lax.bitcast_convert_type(
                    lax.shift_left(v, 16), jnp.float32)
                xb = lax.bitcast_convert_type(
                    jnp.bitwise_and(v, jnp.int32(-65536)), jnp.float32)
                e0 = xa - dec_buf[s, r, pl.ds(g * L, L)]
                e1 = xb - dec_buf[s, r, pl.ds(HW + g * L, L)]
                acc = acc + e0 * e0
                acc = acc + e1 * e1
            return acc

        return lax.fori_loop(0, CH, row_body, acc)

    start_g(0, 0)
    start_g(1, 1)

    def pair_body(t, acc):
        c0 = 2 * t
        wait(c0, 0)
        acc = compute(0, acc)

        @pl.when(t + 1 < NCH // 2)
        def _():
            start(c0 + 2, 0)

        wait(c0 + 1, 1)
        acc = compute(1, acc)

        @pl.when(t + 1 < NCH // 2)
        def _():
            start(c0 + 3, 1)

        return acc

    acc = lax.fori_loop(0, NCH // 2, pair_body,
                        jnp.zeros((L,), jnp.float32))

    acc_buf[...] = acc
    pltpu.sync_copy(acc_buf, out_hbm.at[wid])


_PACK_ROWS = 256


def _tc_pack_body(x_ref, o_ref):
    a = x_ref[:, : DIM // 2]
    b = x_ref[:, DIM // 2:]
    au = lax.bitcast_convert_type(
        a.astype(jnp.bfloat16), jnp.uint16).astype(jnp.uint32)
    bu = lax.bitcast_convert_type(
        b.astype(jnp.bfloat16), jnp.uint16).astype(jnp.uint32)
    o_ref[...] = lax.bitcast_convert_type(
        au | lax.shift_left(bu, jnp.uint32(16)), jnp.int32)


_tc_pack = pl.pallas_call(
    _tc_pack_body,
    grid=(B // _PACK_ROWS,),
    in_specs=[pl.BlockSpec((_PACK_ROWS, DIM), lambda i: (i, 0))],
    out_specs=pl.BlockSpec((_PACK_ROWS, DIM // 2), lambda i: (i, 0)),
    out_shape=jax.ShapeDtypeStruct((B, DIM // 2), jnp.int32),
)


def _tc_finish_body(p_ref, o_ref):
    o_ref[0, 0] = jnp.sqrt(jnp.sum(p_ref[...])) / B


_tc_finish = pl.pallas_call(
    _tc_finish_body,
    out_shape=jax.ShapeDtypeStruct((1, 1), jnp.float32),
    out_specs=pl.BlockSpec(memory_space=pltpu.SMEM),
)


def kernel(inputs, decoded, labels):
    dec_flat = decoded.reshape(B * K, DIM)
    lbl = labels.astype(jnp.int32)
    x_packed = _tc_pack(inputs).reshape(B * DIM // 2)
    partials = _sc_partial_sums(x_packed, dec_flat, lbl)
    return _tc_finish(partials)[0, 0]


# speedup vs baseline: 3.1324x; 1.5535x over previous
import functools

import jax
import jax.numpy as jnp
from jax import lax
from jax.experimental import pallas as pl
from jax.experimental.pallas import tpu as pltpu
from jax.experimental.pallas import tpu_sc as plsc

B = 4096
K = 16
DIM = 1024

NC = 2
NS = 16
NW = NC * NS
L = 16

BPW = B // NW
CH = 16
NCH = BPW // CH
VPR = DIM // L
NBUF = 2

_mesh = plsc.VectorSubcoreMesh(core_axis_name="c", subcore_axis_name="s")


@functools.partial(
    pl.kernel,
    out_type=jax.ShapeDtypeStruct((NW, L), jnp.float32),
    mesh=_mesh,
    scratch_types=[
        pltpu.VMEM((BPW,), jnp.int32),
        pltpu.VMEM((BPW,), jnp.int32),
        pltpu.VMEM((NBUF, CH, DIM), jnp.float32),
        pltpu.VMEM((NBUF, CH, DIM), jnp.float32),
        pltpu.VMEM((L,), jnp.float32),
        [pltpu.SemaphoreType.DMA] * NBUF,
        [pltpu.SemaphoreType.DMA] * NBUF,
    ],
)
def _sc_partial_sums(in_hbm, dec_hbm, lbl_hbm, out_hbm,
                     lbl_v, idx_v, dec_buf, in_buf, acc_buf, sg, si):
    wid = lax.axis_index("s") * NC + lax.axis_index("c")
    base = wid * BPW

    def start_in(c, s):
        pltpu.async_copy(
            in_hbm.at[pl.ds(base + c * CH, CH)], in_buf.at[s], si[s])

    start_in(0, 0)
    start_in(1, 1)

    pltpu.sync_copy(lbl_hbm.at[pl.ds(base, BPW)], lbl_v)
    lane = lax.iota(jnp.int32, L)
    for c in range(BPW // L):
        lbl = lbl_v[pl.ds(c * L, L)]
        idx_v[pl.ds(c * L, L)] = (base + c * L) * K + lane * K + lbl

    def start_g(c, s):
        pltpu.async_copy(
            dec_hbm.at[idx_v.at[pl.ds(c * CH, CH)]], dec_buf.at[s], sg[s])

    def start(c, s):
        start_g(c, s)
        start_in(c, s)

    def wait(c, s):
        pltpu.make_async_copy(
            dec_hbm.at[idx_v.at[pl.ds(c * CH, CH)]], dec_buf.at[s],
            sg[s]).wait()
        pltpu.make_async_copy(
            in_hbm.at[pl.ds(base + c * CH, CH)], in_buf.at[s],
            si[s]).wait()

    def compute(s, acc):
        def row_body(r, acc):
            for v in range(VPR):
                d = (dec_buf[s, r, pl.ds(v * L, L)]
                     - in_buf[s, r, pl.ds(v * L, L)])
                acc = acc + d * d
            return acc

        return lax.fori_loop(0, CH, row_body, acc)

    start_g(0, 0)
    start_g(1, 1)

    def pair_body(t, acc):
        c0 = 2 * t
        wait(c0, 0)
        acc = compute(0, acc)

        @pl.when(t + 1 < NCH // 2)
        def _():
            start(c0 + 2, 0)

        wait(c0 + 1, 1)
        acc = compute(1, acc)

        @pl.when(t + 1 < NCH // 2)
        def _():
            start(c0 + 3, 1)

        return acc

    acc = lax.fori_loop(0, NCH // 2, pair_body,
                        jnp.zeros((L,), jnp.float32))

    acc_buf[...] = acc
    pltpu.sync_copy(acc_buf, out_hbm.at[wid])


def _tc_finish_body(p_ref, o_ref):
    o_ref[0, 0] = jnp.sqrt(jnp.sum(p_ref[...])) / B


_tc_finish = pl.pallas_call(
    _tc_finish_body,
    out_shape=jax.ShapeDtypeStruct((1, 1), jnp.float32),
    out_specs=pl.BlockSpec(memory_space=pltpu.SMEM),
)


def kernel(inputs, decoded, labels):
    dec_flat = decoded.reshape(B * K, DIM)
    lbl = labels.astype(jnp.int32)
    partials = _sc_partial_sums(inputs, dec_flat, lbl)
    return _tc_finish(partials)[0, 0]


# speedup vs baseline: 3.2434x; 1.0354x over previous
import functools

import jax
import jax.numpy as jnp
from jax import lax
from jax.experimental import pallas as pl
from jax.experimental.pallas import tpu as pltpu
from jax.experimental.pallas import tpu_sc as plsc

B = 4096
K = 16
DIM = 1024

NC = 2
NS = 16
NW = NC * NS
L = 16

BPW = B // NW
CH = 16
NCH = BPW // CH
VPR = DIM // L
NBUF = 2

_mesh = plsc.VectorSubcoreMesh(core_axis_name="c", subcore_axis_name="s")


@functools.partial(
    pl.kernel,
    out_type=jax.ShapeDtypeStruct((NW, L), jnp.float32),
    mesh=_mesh,
    scratch_types=[
        pltpu.VMEM((BPW,), jnp.int32),
        pltpu.VMEM((BPW,), jnp.int32),
        pltpu.VMEM((NBUF, CH, DIM), jnp.float32),
        pltpu.VMEM((NBUF, CH, DIM), jnp.float32),
        pltpu.VMEM((L,), jnp.float32),
        [pltpu.SemaphoreType.DMA] * NBUF,
        [pltpu.SemaphoreType.DMA] * NBUF,
    ],
)
def _sc_partial_sums(in_hbm, dec_hbm, lbl_hbm, out_hbm,
                     lbl_v, idx_v, dec_buf, in_buf, acc_buf, sg, si):
    wid = lax.axis_index("s") * NC + lax.axis_index("c")
    base = wid * BPW

    def start_in(c, s):
        pltpu.async_copy(
            in_hbm.at[pl.ds(base + c * CH, CH)], in_buf.at[s], si[s])

    start_in(0, 0)
    start_in(1, 1)

    pltpu.sync_copy(lbl_hbm.at[pl.ds(base, BPW)], lbl_v)
    lane = lax.iota(jnp.int32, L)
    for c in range(BPW // L):
        lbl = lbl_v[pl.ds(c * L, L)]
        idx_v[pl.ds(c * L, L)] = (base + c * L) * K + lane * K + lbl

    def start_g(c, s):
        pltpu.async_copy(
            dec_hbm.at[idx_v.at[pl.ds(c * CH, CH)]], dec_buf.at[s], sg[s])

    def start(c, s):
        start_g(c, s)
        start_in(c, s)

    def wait(c, s):
        pltpu.make_async_copy(
            dec_hbm.at[idx_v.at[pl.ds(c * CH, CH)]], dec_buf.at[s],
            sg[s]).wait()
        pltpu.make_async_copy(
            in_hbm.at[pl.ds(base + c * CH, CH)], in_buf.at[s],
            si[s]).wait()

    def compute(s, acc):
        def half_body(h, acc):
            r = lax.shift_right_logical(h, 1)
            off = (h & 1) * (DIM // 2)
            for v in range(VPR // 2):
                d = (dec_buf[s, r, pl.ds(off + v * L, L)]
                     - in_buf[s, r, pl.ds(off + v * L, L)])
                acc = acc + d * d
            return acc

        return lax.fori_loop(0, 2 * CH, half_body, acc)

    start_g(0, 0)
    start_g(1, 1)

    def pair_body(t, acc):
        c0 = 2 * t
        wait(c0, 0)
        acc = compute(0, acc)

        @pl.when(t + 1 < NCH // 2)
        def _():
            start(c0 + 2, 0)

        wait(c0 + 1, 1)
        acc = compute(1, acc)

        @pl.when(t + 1 < NCH // 2)
        def _():
            start(c0 + 3, 1)

        return acc

    acc = lax.fori_loop(0, NCH // 2, pair_body,
                        jnp.zeros((L,), jnp.float32))

    acc_buf[...] = acc
    pltpu.sync_copy(acc_buf, out_hbm.at[wid])


def _tc_finish_body(p_ref, o_ref):
    o_ref[0, 0] = jnp.sqrt(jnp.sum(p_ref[...])) / B


_tc_finish = pl.pallas_call(
    _tc_finish_body,
    out_shape=jax.ShapeDtypeStruct((1, 1), jnp.float32),
    out_specs=pl.BlockSpec(memory_space=pltpu.SMEM),
)


def kernel(inputs, decoded, labels):
    dec_flat = decoded.reshape(B * K, DIM)
    lbl = labels.astype(jnp.int32)
    partials = _sc_partial_sums(inputs, dec_flat, lbl)
    return _tc_finish(partials)[0, 0]


# speedup vs baseline: 3.2570x; 1.0042x over previous
import functools

import jax
import jax.numpy as jnp
from jax import lax
from jax.experimental import pallas as pl
from jax.experimental.pallas import tpu as pltpu
from jax.experimental.pallas import tpu_sc as plsc

B = 4096
K = 16
DIM = 1024

NC = 2
NS = 16
NW = NC * NS
L = 16

BPW = B // NW
CH = 16
NCH = BPW // CH
VPR = DIM // L
NBUF = 2

_mesh = plsc.VectorSubcoreMesh(core_axis_name="c", subcore_axis_name="s")


@functools.partial(
    pl.kernel,
    out_type=jax.ShapeDtypeStruct((NW, L), jnp.float32),
    mesh=_mesh,
    scratch_types=[
        pltpu.VMEM((BPW,), jnp.int32),
        pltpu.VMEM((BPW,), jnp.int32),
        pltpu.VMEM((NBUF, CH, DIM), jnp.float32),
        pltpu.VMEM((NBUF, CH, DIM), jnp.float32),
        pltpu.VMEM((L,), jnp.float32),
        [pltpu.SemaphoreType.DMA] * NBUF,
        [pltpu.SemaphoreType.DMA] * NBUF,
    ],
)
def _sc_partial_sums(in_hbm, dec_hbm, lbl_hbm, out_hbm,
                     lbl_v, idx_v, dec_buf, in_buf, acc_buf, sg, si):
    wid = lax.axis_index("s") * NC + lax.axis_index("c")
    base = wid * BPW

    def start_in(c, s):
        pltpu.async_copy(
            in_hbm.at[pl.ds(base + c * CH, CH)], in_buf.at[s], si[s])

    start_in(0, 0)
    start_in(1, 1)

    pltpu.sync_copy(lbl_hbm.at[pl.ds(base, BPW)], lbl_v)
    lane = lax.iota(jnp.int32, L)
    for c in range(BPW // L):
        lbl = lbl_v[pl.ds(c * L, L)]
        idx_v[pl.ds(c * L, L)] = (base + c * L) * K + lane * K + lbl

    def start_g(c, s):
        pltpu.async_copy(
            dec_hbm.at[idx_v.at[pl.ds(c * CH, CH)]], dec_buf.at[s], sg[s])

    def start(c, s):
        start_g(c, s)
        start_in(c, s)

    def wait(c, s):
        pltpu.make_async_copy(
            dec_hbm.at[idx_v.at[pl.ds(c * CH, CH)]], dec_buf.at[s],
            sg[s]).wait()
        pltpu.make_async_copy(
            in_hbm.at[pl.ds(base + c * CH, CH)], in_buf.at[s],
            si[s]).wait()

    def compute(s, acc):
        def quarter_body(h, acc):
            r = lax.shift_right_logical(h, 2)
            off = (h & 3) * (DIM // 4)
            for v in range(VPR // 4):
                d = (dec_buf[s, r, pl.ds(off + v * L, L)]
                     - in_buf[s, r, pl.ds(off + v * L, L)])
                acc = acc + d * d
            return acc

        return lax.fori_loop(0, 4 * CH, quarter_body, acc)

    start_g(0, 0)
    start_g(1, 1)

    def pair_body(t, acc):
        c0 = 2 * t
        wait(c0, 0)
        acc = compute(0, acc)

        @pl.when(t + 1 < NCH // 2)
        def _():
            start(c0 + 2, 0)

        wait(c0 + 1, 1)
        acc = compute(1, acc)

        @pl.when(t + 1 < NCH // 2)
        def _():
            start(c0 + 3, 1)

        return acc

    acc = lax.fori_loop(0, NCH // 2, pair_body,
                        jnp.zeros((L,), jnp.float32))

    acc_buf[...] = acc
    pltpu.sync_copy(acc_buf, out_hbm.at[wid])


def _tc_finish_body(p_ref, o_ref):
    o_ref[0, 0] = jnp.sqrt(jnp.sum(p_ref[...])) / B


_tc_finish = pl.pallas_call(
    _tc_finish_body,
    out_shape=jax.ShapeDtypeStruct((1, 1), jnp.float32),
    out_specs=pl.BlockSpec(memory_space=pltpu.SMEM),
)


def kernel(inputs, decoded, labels):
    dec_flat = decoded.reshape(B * K, DIM)
    lbl = labels.astype(jnp.int32)
    partials = _sc_partial_sums(inputs, dec_flat, lbl)
    return _tc_finish(partials)[0, 0]
